# baseline (speedup 1.0000x reference)
import functools

import jax
import jax.numpy as jnp
import numpy as np
from jax import lax
from jax.experimental import pallas as pl
from jax.experimental.pallas import tpu as pltpu
from jax.experimental.pallas import tpu_sc as plsc

L, B, D = 200, 256, 64
R = L * B
CHUNK = 128
NODE_CHUNKS = R // CHUNK
NW = 32
CPW = NODE_CHUNKS // (NW // 2)


def _pe_table():
    pos = jnp.arange(L, dtype=jnp.float32)[:, None]
    div = jnp.exp(jnp.arange(0, D, 2, dtype=jnp.float32) * (-np.log(10000.0) / D))
    pe = jnp.zeros((L, D), dtype=jnp.float32)
    pe = pe.at[:, 0::2].set(jnp.sin(pos * div))
    pe = pe.at[:, 1::2].set(jnp.cos(pos * div))
    return pe


def _sc_gather(node_idx_h, edge_idx_h, node_tab_h, edge_tab_h, pe_h,
               out_h, idx_v, rows_v, pe_v, sem):
    cid = lax.axis_index("c")
    sid = lax.axis_index("s")
    wid = sid * 2 + cid

    pltpu.sync_copy(pe_h, pe_v)

    def do_chunks(idx_h, tab_h, w, out_row0):
        def body(k, _):
            c = w * CPW + k
            row0 = c * CHUNK
            l = c // 2
            pltpu.sync_copy(idx_h.at[pl.ds(row0, CHUNK)], idx_v)
            pltpu.async_copy(tab_h.at[idx_v], rows_v, sem).wait()

            pe0 = pe_v[pl.ds(l * D, 16)]
            pe1 = pe_v[pl.ds(l * D + 16, 16)]
            pe2 = pe_v[pl.ds(l * D + 32, 16)]
            pe3 = pe_v[pl.ds(l * D + 48, 16)]

            def add_pe(i, _):
                rows_v[i, pl.ds(0, 16)] = rows_v[i, pl.ds(0, 16)] + pe0
                rows_v[i, pl.ds(16, 16)] = rows_v[i, pl.ds(16, 16)] + pe1
                rows_v[i, pl.ds(32, 16)] = rows_v[i, pl.ds(32, 16)] + pe2
                rows_v[i, pl.ds(48, 16)] = rows_v[i, pl.ds(48, 16)] + pe3
                return 0

            lax.fori_loop(0, CHUNK, add_pe, 0)
            pltpu.sync_copy(rows_v, out_h.at[pl.ds(out_row0 + row0, CHUNK)])
            return 0

        lax.fori_loop(0, CPW, body, 0)

    @pl.when(wid < 16)
    def _():
        do_chunks(node_idx_h, node_tab_h, wid, 0)

    @pl.when(wid >= 16)
    def _():
        do_chunks(edge_idx_h, edge_tab_h, wid - 16, R)


def _make_sc_call():
    mesh = plsc.VectorSubcoreMesh(core_axis_name="c", subcore_axis_name="s")
    return pl.kernel(
        _sc_gather,
        out_type=jax.ShapeDtypeStruct((3 * R, D), jnp.float32),
        mesh=mesh,
        compiler_params=pltpu.CompilerParams(use_tc_tiling_on_sc=False),
        scratch_types=[
            pltpu.VMEM((CHUNK,), jnp.int32),
            pltpu.VMEM((CHUNK, D), jnp.float32),
            pltpu.VMEM((L * D,), jnp.float32),
            pltpu.SemaphoreType.DMA,
        ],
    )


def _mm_body(alias_ref, x_ref, w_ref, pe_ref, out_ref):
    del alias_ref
    i = pl.program_id(0)
    prod = jnp.dot(x_ref[...], w_ref[...], preferred_element_type=jnp.float32)
    out_ref[...] = prod + pe_ref[pl.ds(i, 1), :]


def _mm_call(sc_out, x, w, pe):
    grid = (L,)
    return pl.pallas_call(
        _mm_body,
        grid=grid,
        in_specs=[
            pl.BlockSpec(memory_space=pl.ANY),
            pl.BlockSpec((B, 1000), lambda i: (i, 0)),
            pl.BlockSpec((1000, D), lambda i: (0, 0)),
            pl.BlockSpec((L, D), lambda i: (0, 0)),
        ],
        out_specs=pl.BlockSpec((B, D), lambda i: (2 * L + i, 0)),
        out_shape=jax.ShapeDtypeStruct((3 * R, D), jnp.float32),
        input_output_aliases={0: 0},
    )(sc_out, x, w, pe)


def kernel(node_idx, edge_idx, node_val_mat, node_embed_table,
           edge_embed_table, val_tok_embed):
    pe = _pe_table()
    sc_out = _make_sc_call()(
        node_idx.reshape(-1).astype(jnp.int32),
        edge_idx.reshape(-1).astype(jnp.int32),
        node_embed_table,
        edge_embed_table,
        pe.reshape(-1),
    )
    out = _mm_call(sc_out, node_val_mat, val_tok_embed, pe)
    return out.reshape(3 * L, B, D)
